# baseline (speedup 1.0000x reference)
import functools

import jax
import jax.numpy as jnp
from jax import lax
from jax.experimental import pallas as pl
from jax.experimental.pallas import tpu as pltpu
from jax.experimental.pallas import tpu_sc as plsc

NE = 8
NC = 2
NS = 16
NW = NC * NS
TB = 1024
BS = 256


def _gate_body(x_ref, gw_ref, gb_ref, e12_ref, r12_ref, cnt_ref, xbf_ref,
               carry):
    i = pl.program_id(0)
    nb = pl.num_programs(0)

    @pl.when(i == 0)
    def _():
        carry[...] = jnp.zeros_like(carry)

    x = x_ref[...]
    xbf_ref[...] = x.astype(jnp.bfloat16)
    logits = jnp.dot(x, gw_ref[...].T, preferred_element_type=jnp.float32)
    logits = logits + gb_ref[...]
    m1 = jnp.max(logits, axis=-1, keepdims=True)
    l2 = jnp.where(logits >= m1, -jnp.inf, logits)
    m2 = jnp.max(l2, axis=-1, keepdims=True)
    top2 = logits >= m2
    eids = lax.broadcasted_iota(jnp.int32, logits.shape, 1)
    big = jnp.int32(1 << 20)
    e1 = jnp.min(jnp.where(logits >= m1, eids, big), axis=-1, keepdims=True)
    e2 = jnp.min(jnp.where(top2 & (logits < m1), eids, big), axis=-1,
                 keepdims=True)

    maskf = top2.astype(jnp.float32)
    ri = lax.broadcasted_iota(jnp.int32, (TB, TB), 0)
    ci = lax.broadcasted_iota(jnp.int32, (TB, TB), 1)
    tril = jnp.where(ri > ci, 1.0, 0.0)
    excl = jnp.dot(tril, maskf, preferred_element_type=jnp.float32)
    rank_mat = carry[...] + excl
    r1 = jnp.sum(jnp.where(eids == e1, rank_mat, 0.0), axis=1, keepdims=True)
    r2 = jnp.sum(jnp.where(eids == e2, rank_mat, 0.0), axis=1, keepdims=True)

    e12_ref[...] = jnp.concatenate([e1, e2], axis=1)
    r12_ref[...] = jnp.concatenate([r1, r2], axis=1).astype(jnp.int32)

    new_carry = carry[...] + jnp.sum(maskf, axis=0, keepdims=True)
    carry[...] = new_carry

    @pl.when(i == nb - 1)
    def _():
        cnt_ref[...] = new_carry.astype(jnp.int32)


def _gating(xf, gate_W, gb2):
    M, D = xf.shape
    return pl.pallas_call(
        _gate_body,
        grid=(M // TB,),
        in_specs=[
            pl.BlockSpec((TB, D), lambda i: (i, 0)),
            pl.BlockSpec((NE, D), lambda i: (0, 0)),
            pl.BlockSpec((1, NE), lambda i: (0, 0)),
        ],
        out_specs=[
            pl.BlockSpec((TB, 2), lambda i: (i, 0)),
            pl.BlockSpec((TB, 2), lambda i: (i, 0)),
            pl.BlockSpec((1, NE), lambda i: (0, 0)),
            pl.BlockSpec((TB, D), lambda i: (i, 0)),
        ],
        out_shape=[
            jax.ShapeDtypeStruct((M, 2), jnp.int32),
            jax.ShapeDtypeStruct((M, 2), jnp.int32),
            jax.ShapeDtypeStruct((1, NE), jnp.int32),
            jax.ShapeDtypeStruct((M, D), jnp.bfloat16),
        ],
        scratch_shapes=[pltpu.VMEM((1, NE), jnp.float32)],
        compiler_params=pltpu.CompilerParams(
            dimension_semantics=("arbitrary",)),
    )(xf, gate_W, gb2)


def _make_dispatch(M, D, S_pad, n_chunk):
    mesh = plsc.VectorSubcoreMesh(core_axis_name="c", subcore_axis_name="s",
                                  num_cores=NC, num_subcores=NS)
    per_w = M // NW
    C = per_w // n_chunk
    SL = D // 2 // 128

    @functools.partial(
        pl.kernel, mesh=mesh,
        out_type=jax.ShapeDtypeStruct((S_pad, SL, 128), jnp.int32),
        scratch_types=[
            pltpu.VMEM((C, SL, 128), jnp.int32),
            pltpu.VMEM((C, SL, 128), jnp.int32),
            pltpu.VMEM((n_chunk, C), jnp.int32),
            pltpu.VMEM((n_chunk, C), jnp.int32),
            pltpu.SemaphoreType.DMA,
            pltpu.SemaphoreType.DMA,
            pltpu.SemaphoreType.DMA,
            pltpu.SemaphoreType.DMA,
        ],
    )
    def dispatch(x_hbm, s1_hbm, s2_hbm, xs_hbm, xb0, xb1, i1, i2,
                 sl0, sl1, ss0, ss1):
        wid = lax.axis_index("s") * NC + lax.axis_index("c")
        base = wid * per_w
        pltpu.sync_copy(s1_hbm.at[wid], i1)
        pltpu.sync_copy(s2_hbm.at[wid], i2)
        xb = [xb0, xb1]
        sem_l = [sl0, sl1]
        sem_s = [ss0, ss1]
        ld = [None] * n_chunk
        sc = [None] * (2 * n_chunk)
        ld[0] = pltpu.async_copy(x_hbm.at[pl.ds(base, C)], xb[0], sem_l[0])
        for c in range(n_chunk):
            b = c & 1
            if c + 1 < n_chunk:
                b2 = (c + 1) & 1
                if c >= 1:
                    sc[2 * (c - 1)].wait()
                    sc[2 * (c - 1) + 1].wait()
                ld[c + 1] = pltpu.async_copy(
                    x_hbm.at[pl.ds(base + (c + 1) * C, C)], xb[b2],
                    sem_l[b2])
            ld[c].wait()
            sc[2 * c] = pltpu.async_copy(xb[b], xs_hbm.at[i1.at[c]],
                                         sem_s[b])
            sc[2 * c + 1] = pltpu.async_copy(xb[b], xs_hbm.at[i2.at[c]],
                                             sem_s[b])
        for c in (n_chunk - 2, n_chunk - 1):
            sc[2 * c].wait()
            sc[2 * c + 1].wait()

    return dispatch


def _gmm_body(be_ref, xs_ref, w_ref, b_ref, y_ref):
    y = lax.dot_general(
        xs_ref[...], w_ref[0],
        (((1,), (1,)), ((), ())), preferred_element_type=jnp.float32)
    y_ref[...] = y + b_ref[0]


def _grouped_matmul(xs, wq, eb3, be):
    S_pad, D = xs.shape
    O = wq.shape[1]
    nb = S_pad // BS
    grid_spec = pltpu.PrefetchScalarGridSpec(
        num_scalar_prefetch=1,
        grid=(nb,),
        in_specs=[
            pl.BlockSpec((BS, D), lambda i, be: (i, 0)),
            pl.BlockSpec((1, O, D), lambda i, be: (be[i], 0, 0)),
            pl.BlockSpec((1, 1, O), lambda i, be: (be[i], 0, 0)),
        ],
        out_specs=pl.BlockSpec((BS, O), lambda i, be: (i, 0)),
    )
    return pl.pallas_call(
        _gmm_body,
        grid_spec=grid_spec,
        out_shape=jax.ShapeDtypeStruct((S_pad, O), jnp.float32),
        compiler_params=pltpu.CompilerParams(
            dimension_semantics=("arbitrary",)),
    )(be, xs, wq, eb3)


def _make_combine(M, O, S_pad, n_chunk):
    mesh = plsc.VectorSubcoreMesh(core_axis_name="c", subcore_axis_name="s",
                                  num_cores=NC, num_subcores=NS)
    per_w = M // NW
    C = per_w // n_chunk
    L = 16

    @functools.partial(
        pl.kernel, mesh=mesh,
        out_type=jax.ShapeDtypeStruct((M, O), jnp.float32),
        scratch_types=[
            pltpu.VMEM((C, O), jnp.float32),
            pltpu.VMEM((C, O), jnp.float32),
            pltpu.VMEM((C, O), jnp.float32),
            pltpu.VMEM((C, O), jnp.float32),
            pltpu.VMEM((n_chunk, C), jnp.int32),
            pltpu.VMEM((n_chunk, C), jnp.int32),
            pltpu.SemaphoreType.DMA,
            pltpu.SemaphoreType.DMA,
            pltpu.SemaphoreType.DMA,
            pltpu.SemaphoreType.DMA,
        ],
    )
    def combine(ys_hbm, s1_hbm, s2_hbm, out_hbm, b1a, b2a, b1b, b2b, i1, i2,
                sg0, sg1, sw0, sw1):
        wid = lax.axis_index("s") * NC + lax.axis_index("c")
        base = wid * per_w
        pltpu.sync_copy(s1_hbm.at[wid], i1)
        pltpu.sync_copy(s2_hbm.at[wid], i2)
        b1 = [b1a, b1b]
        b2 = [b2a, b2b]
        sem_g = [sg0, sg1]
        sem_w = [sw0, sw1]
        g = [None] * (2 * n_chunk)
        w = [None] * n_chunk
        g[0] = pltpu.async_copy(ys_hbm.at[i1.at[0]], b1[0], sem_g[0])
        g[1] = pltpu.async_copy(ys_hbm.at[i2.at[0]], b2[0], sem_g[0])
        for c in range(n_chunk):
            s = c & 1
            if c + 1 < n_chunk:
                s2 = (c + 1) & 1
                if c >= 1:
                    w[c - 1].wait()
                g[2 * (c + 1)] = pltpu.async_copy(
                    ys_hbm.at[i1.at[c + 1]], b1[s2], sem_g[s2])
                g[2 * (c + 1) + 1] = pltpu.async_copy(
                    ys_hbm.at[i2.at[c + 1]], b2[s2], sem_g[s2])
            g[2 * c].wait()
            g[2 * c + 1].wait()

            def add_cols(j, _, s=s):
                for r in range(C):
                    b1[s][r, pl.ds(j * L, L)] = (
                        b1[s][r, pl.ds(j * L, L)]
                        + b2[s][r, pl.ds(j * L, L)])
                return 0

            lax.fori_loop(0, O // L, add_cols, 0)
            w[c] = pltpu.async_copy(b1[s], out_hbm.at[pl.ds(base + c * C, C)],
                                    sem_w[s])
        w[n_chunk - 1].wait()

    return combine


def kernel(x, gate_W, gate_b, expert_W, expert_b):
    orig_shape = x.shape
    D = x.shape[-1]
    M = x.size // D
    O = expert_W.shape[1]
    SL = D // 128
    xf = x.reshape(M, D)
    gb2 = gate_b.reshape(1, NE)
    wq = expert_W.astype(jnp.bfloat16)
    eb3 = expert_b.reshape(NE, 1, O)

    nb_max = M * 2 // BS + (NE - 1)
    S_pad = nb_max * BS
    n_disp, n_comb = 8, 32
    per_w = M // NW

    e12, r12, cnt, xbf = _gating(xf, gate_W, gb2)

    counts = cnt[0]
    padded = ((counts + BS - 1) // BS) * BS
    poff = jnp.concatenate([jnp.zeros((1,), jnp.int32),
                            jnp.cumsum(padded)[:-1].astype(jnp.int32)])
    slot = jnp.take(poff, e12, axis=0) + r12
    s1 = slot[:, 0].reshape(NW, n_disp, per_w // n_disp)
    s2 = slot[:, 1].reshape(NW, n_disp, per_w // n_disp)
    starts = jnp.arange(nb_max, dtype=jnp.int32) * BS
    be = (jnp.sum(starts[:, None] >= poff[None, :], axis=1) - 1).astype(
        jnp.int32)
    be = jnp.clip(be, 0, NE - 1)

    xi = lax.bitcast_convert_type(xbf.reshape(M, D // 2, 2), jnp.int32)
    xi3 = xi.reshape(M, D // 2 // 128, 128)
    xs_i = _make_dispatch(M, D, S_pad, n_disp)(xi3, s1, s2)
    xs = lax.bitcast_convert_type(
        xs_i.reshape(S_pad, D // 2), jnp.bfloat16).reshape(S_pad, D)
    ys = _grouped_matmul(xs, wq, eb3, be)
    s1c = slot[:, 0].reshape(NW, n_comb, per_w // n_comb)
    s2c = slot[:, 1].reshape(NW, n_comb, per_w // n_comb)
    out = _make_combine(M, O, S_pad, n_comb)(ys, s1c, s2c)
    return out.reshape(orig_shape[:-1] + (O,))


# speedup vs baseline: 2.8050x; 2.8050x over previous
import functools

import jax
import jax.numpy as jnp
from jax import lax
from jax.experimental import pallas as pl
from jax.experimental.pallas import tpu as pltpu
from jax.experimental.pallas import tpu_sc as plsc

NE = 8
NC = 2
NS = 16
NW = NC * NS
TB = 1024
BS = 256


def _gate_body(x_ref, gw_ref, gb_ref, e12_ref, r12_ref, cnt_ref, xq_ref,
               carry):
    i = pl.program_id(0)
    nb = pl.num_programs(0)

    @pl.when(i == 0)
    def _():
        carry[...] = jnp.zeros_like(carry)

    x = x_ref[...]
    d2 = x.shape[1] // 2
    lo = x[:, :d2].astype(jnp.bfloat16).astype(jnp.float32)
    hi = x[:, d2:].astype(jnp.bfloat16).astype(jnp.float32)
    lo_i = lax.bitcast_convert_type(lo, jnp.int32)
    hi_i = lax.shift_right_logical(lax.bitcast_convert_type(hi, jnp.int32),
                                   16)
    xq_ref[...] = lo_i | hi_i
    logits = jnp.dot(x, gw_ref[...].T, preferred_element_type=jnp.float32)
    logits = logits + gb_ref[...]
    m1 = jnp.max(logits, axis=-1, keepdims=True)
    l2 = jnp.where(logits >= m1, -jnp.inf, logits)
    m2 = jnp.max(l2, axis=-1, keepdims=True)
    top2 = logits >= m2
    eids = lax.broadcasted_iota(jnp.int32, logits.shape, 1)
    big = jnp.int32(1 << 20)
    e1 = jnp.min(jnp.where(logits >= m1, eids, big), axis=-1, keepdims=True)
    e2 = jnp.min(jnp.where(top2 & (logits < m1), eids, big), axis=-1,
                 keepdims=True)

    maskf = top2.astype(jnp.float32)
    ri = lax.broadcasted_iota(jnp.int32, (TB, TB), 0)
    ci = lax.broadcasted_iota(jnp.int32, (TB, TB), 1)
    tril = jnp.where(ri > ci, 1.0, 0.0)
    excl = jnp.dot(tril, maskf, preferred_element_type=jnp.float32)
    rank_mat = carry[...] + excl
    r1 = jnp.sum(jnp.where(eids == e1, rank_mat, 0.0), axis=1, keepdims=True)
    r2 = jnp.sum(jnp.where(eids == e2, rank_mat, 0.0), axis=1, keepdims=True)

    e12_ref[...] = jnp.concatenate([e1, e2], axis=1)
    r12_ref[...] = jnp.concatenate([r1, r2], axis=1).astype(jnp.int32)

    new_carry = carry[...] + jnp.sum(maskf, axis=0, keepdims=True)
    carry[...] = new_carry

    @pl.when(i == nb - 1)
    def _():
        cnt_ref[...] = new_carry.astype(jnp.int32)


def _gating(xf, gate_W, gb2):
    M, D = xf.shape
    return pl.pallas_call(
        _gate_body,
        grid=(M // TB,),
        in_specs=[
            pl.BlockSpec((TB, D), lambda i: (i, 0)),
            pl.BlockSpec((NE, D), lambda i: (0, 0)),
            pl.BlockSpec((1, NE), lambda i: (0, 0)),
        ],
        out_specs=[
            pl.BlockSpec((TB, 2), lambda i: (i, 0)),
            pl.BlockSpec((TB, 2), lambda i: (i, 0)),
            pl.BlockSpec((1, NE), lambda i: (0, 0)),
            pl.BlockSpec((TB, D // 2), lambda i: (i, 0)),
        ],
        out_shape=[
            jax.ShapeDtypeStruct((M, 2), jnp.int32),
            jax.ShapeDtypeStruct((M, 2), jnp.int32),
            jax.ShapeDtypeStruct((1, NE), jnp.int32),
            jax.ShapeDtypeStruct((M, D // 2), jnp.int32),
        ],
        scratch_shapes=[pltpu.VMEM((1, NE), jnp.float32)],
        compiler_params=pltpu.CompilerParams(
            dimension_semantics=("arbitrary",)),
    )(xf, gate_W, gb2)


def _make_dispatch(M, D, S_pad, n_chunk):
    mesh = plsc.VectorSubcoreMesh(core_axis_name="c", subcore_axis_name="s",
                                  num_cores=NC, num_subcores=NS)
    per_w = M // NW
    C = per_w // n_chunk
    SL = D // 2 // 128

    @functools.partial(
        pl.kernel, mesh=mesh,
        out_type=jax.ShapeDtypeStruct((S_pad, SL, 128), jnp.int32),
        scratch_types=[
            pltpu.VMEM((C, SL, 128), jnp.int32),
            pltpu.VMEM((C, SL, 128), jnp.int32),
            pltpu.VMEM((n_chunk, C), jnp.int32),
            pltpu.VMEM((n_chunk, C), jnp.int32),
            pltpu.SemaphoreType.DMA,
            pltpu.SemaphoreType.DMA,
            pltpu.SemaphoreType.DMA,
            pltpu.SemaphoreType.DMA,
        ],
    )
    def dispatch(x_hbm, s1_hbm, s2_hbm, xs_hbm, xb0, xb1, i1, i2,
                 sl0, sl1, ss0, ss1):
        wid = lax.axis_index("s") * NC + lax.axis_index("c")
        base = wid * per_w
        pltpu.sync_copy(s1_hbm.at[wid], i1)
        pltpu.sync_copy(s2_hbm.at[wid], i2)
        xb = [xb0, xb1]
        sem_l = [sl0, sl1]
        sem_s = [ss0, ss1]
        ld = [None] * n_chunk
        sc = [None] * (2 * n_chunk)
        ld[0] = pltpu.async_copy(x_hbm.at[pl.ds(base, C)], xb[0], sem_l[0])
        for c in range(n_chunk):
            b = c & 1
            if c + 1 < n_chunk:
                b2 = (c + 1) & 1
                if c >= 1:
                    sc[2 * (c - 1)].wait()
                    sc[2 * (c - 1) + 1].wait()
                ld[c + 1] = pltpu.async_copy(
                    x_hbm.at[pl.ds(base + (c + 1) * C, C)], xb[b2],
                    sem_l[b2])
            ld[c].wait()
            sc[2 * c] = pltpu.async_copy(xb[b], xs_hbm.at[i1.at[c]],
                                         sem_s[b])
            sc[2 * c + 1] = pltpu.async_copy(xb[b], xs_hbm.at[i2.at[c]],
                                             sem_s[b])
        for c in (n_chunk - 2, n_chunk - 1):
            sc[2 * c].wait()
            sc[2 * c + 1].wait()

    return dispatch


def _gmm_body(be_ref, xs_ref, w_ref, b_ref, y_ref):
    w = xs_ref[...]
    lo = lax.bitcast_convert_type(w & jnp.int32(-65536), jnp.float32)
    hi = lax.bitcast_convert_type(lax.shift_left(w, 16), jnp.float32)
    xb = jnp.concatenate([lo, hi], axis=1).astype(jnp.bfloat16)
    y = lax.dot_general(
        xb, w_ref[0],
        (((1,), (1,)), ((), ())), preferred_element_type=jnp.float32)
    y_ref[...] = y + b_ref[0]


def _grouped_matmul(xs, wq, eb3, be):
    S_pad, D2 = xs.shape
    O = wq.shape[1]
    nb = S_pad // BS
    grid_spec = pltpu.PrefetchScalarGridSpec(
        num_scalar_prefetch=1,
        grid=(nb,),
        in_specs=[
            pl.BlockSpec((BS, D2), lambda i, be: (i, 0)),
            pl.BlockSpec((1, O, 2 * D2), lambda i, be: (be[i], 0, 0)),
            pl.BlockSpec((1, 1, O), lambda i, be: (be[i], 0, 0)),
        ],
        out_specs=pl.BlockSpec((BS, O), lambda i, be: (i, 0)),
    )
    return pl.pallas_call(
        _gmm_body,
        grid_spec=grid_spec,
        out_shape=jax.ShapeDtypeStruct((S_pad, O), jnp.float32),
        compiler_params=pltpu.CompilerParams(
            dimension_semantics=("arbitrary",)),
    )(be, xs, wq, eb3)


def _make_combine(M, O, S_pad, n_chunk):
    mesh = plsc.VectorSubcoreMesh(core_axis_name="c", subcore_axis_name="s",
                                  num_cores=NC, num_subcores=NS)
    per_w = M // NW
    C = per_w // n_chunk
    L = 16

    @functools.partial(
        pl.kernel, mesh=mesh,
        out_type=jax.ShapeDtypeStruct((M, O), jnp.float32),
        scratch_types=[
            pltpu.VMEM((C, O), jnp.float32),
            pltpu.VMEM((C, O), jnp.float32),
            pltpu.VMEM((C, O), jnp.float32),
            pltpu.VMEM((C, O), jnp.float32),
            pltpu.VMEM((n_chunk, C), jnp.int32),
            pltpu.VMEM((n_chunk, C), jnp.int32),
            pltpu.SemaphoreType.DMA,
            pltpu.SemaphoreType.DMA,
            pltpu.SemaphoreType.DMA,
            pltpu.SemaphoreType.DMA,
        ],
    )
    def combine(ys_hbm, s1_hbm, s2_hbm, out_hbm, b1a, b2a, b1b, b2b, i1, i2,
                sg0, sg1, sw0, sw1):
        wid = lax.axis_index("s") * NC + lax.axis_index("c")
        base = wid * per_w
        pltpu.sync_copy(s1_hbm.at[wid], i1)
        pltpu.sync_copy(s2_hbm.at[wid], i2)
        b1 = [b1a, b1b]
        b2 = [b2a, b2b]
        sem_g = [sg0, sg1]
        sem_w = [sw0, sw1]
        g = [None] * (2 * n_chunk)
        w = [None] * n_chunk
        g[0] = pltpu.async_copy(ys_hbm.at[i1.at[0]], b1[0], sem_g[0])
        g[1] = pltpu.async_copy(ys_hbm.at[i2.at[0]], b2[0], sem_g[0])
        for c in range(n_chunk):
            s = c & 1
            if c + 1 < n_chunk:
                s2 = (c + 1) & 1
                if c >= 1:
                    w[c - 1].wait()
                g[2 * (c + 1)] = pltpu.async_copy(
                    ys_hbm.at[i1.at[c + 1]], b1[s2], sem_g[s2])
                g[2 * (c + 1) + 1] = pltpu.async_copy(
                    ys_hbm.at[i2.at[c + 1]], b2[s2], sem_g[s2])
            g[2 * c].wait()
            g[2 * c + 1].wait()

            def add_cols(j, _, s=s):
                for r in range(C):
                    b1[s][r, pl.ds(j * L, L)] = (
                        b1[s][r, pl.ds(j * L, L)]
                        + b2[s][r, pl.ds(j * L, L)])
                return 0

            lax.fori_loop(0, O // L, add_cols, 0)
            w[c] = pltpu.async_copy(b1[s], out_hbm.at[pl.ds(base + c * C, C)],
                                    sem_w[s])
        w[n_chunk - 1].wait()

    return combine


def kernel(x, gate_W, gate_b, expert_W, expert_b):
    orig_shape = x.shape
    D = x.shape[-1]
    M = x.size // D
    O = expert_W.shape[1]
    SL = D // 128
    xf = x.reshape(M, D)
    gb2 = gate_b.reshape(1, NE)
    wq = expert_W.astype(jnp.bfloat16)
    eb3 = expert_b.reshape(NE, 1, O)

    nb_max = M * 2 // BS + (NE - 1)
    S_pad = nb_max * BS
    n_disp, n_comb = 8, 32
    per_w = M // NW

    e12, r12, cnt, xq = _gating(xf, gate_W, gb2)

    counts = cnt[0]
    padded = ((counts + BS - 1) // BS) * BS
    poff = jnp.concatenate([jnp.zeros((1,), jnp.int32),
                            jnp.cumsum(padded)[:-1].astype(jnp.int32)])
    slot = jnp.take(poff, e12, axis=0) + r12
    s1 = slot[:, 0].reshape(NW, n_disp, per_w // n_disp)
    s2 = slot[:, 1].reshape(NW, n_disp, per_w // n_disp)
    starts = jnp.arange(nb_max, dtype=jnp.int32) * BS
    be = (jnp.sum(starts[:, None] >= poff[None, :], axis=1) - 1).astype(
        jnp.int32)
    be = jnp.clip(be, 0, NE - 1)

    xq3 = xq.reshape(M, D // 2 // 128, 128)
    xs_i = _make_dispatch(M, D, S_pad, n_disp)(xq3, s1, s2)
    ys = _grouped_matmul(xs_i.reshape(S_pad, D // 2), wq, eb3, be)
    s1c = slot[:, 0].reshape(NW, n_comb, per_w // n_comb)
    s2c = slot[:, 1].reshape(NW, n_comb, per_w // n_comb)
    out = _make_combine(M, O, S_pad, n_comb)(ys, s1c, s2c)
    return out.reshape(orig_shape[:-1] + (O,))


# speedup vs baseline: 3.2117x; 1.1450x over previous
import functools

import jax
import jax.numpy as jnp
from jax import lax
from jax.experimental import pallas as pl
from jax.experimental.pallas import tpu as pltpu
from jax.experimental.pallas import tpu_sc as plsc

NE = 8
NC = 2
NS = 16
NW = NC * NS
TB = 1024
BS = 256


def _gate_body(x_ref, gw_ref, gb_ref, e12_ref, r12_ref, cnt_ref, xq_ref,
               carry):
    i = pl.program_id(0)
    nb = pl.num_programs(0)

    @pl.when(i == 0)
    def _():
        carry[...] = jnp.zeros_like(carry)

    x = x_ref[...]
    d2 = x.shape[1] // 2
    lo = x[:, :d2].astype(jnp.bfloat16).astype(jnp.float32)
    hi = x[:, d2:].astype(jnp.bfloat16).astype(jnp.float32)
    lo_i = lax.bitcast_convert_type(lo, jnp.int32)
    hi_i = lax.shift_right_logical(lax.bitcast_convert_type(hi, jnp.int32),
                                   16)
    xq_ref[...] = lo_i | hi_i
    logits = jnp.dot(x, gw_ref[...].T, preferred_element_type=jnp.float32)
    logits = logits + gb_ref[...]
    m1 = jnp.max(logits, axis=-1, keepdims=True)
    l2 = jnp.where(logits >= m1, -jnp.inf, logits)
    m2 = jnp.max(l2, axis=-1, keepdims=True)
    top2 = logits >= m2
    eids = lax.broadcasted_iota(jnp.int32, logits.shape, 1)
    big = jnp.int32(1 << 20)
    e1 = jnp.min(jnp.where(logits >= m1, eids, big), axis=-1, keepdims=True)
    e2 = jnp.min(jnp.where(top2 & (logits < m1), eids, big), axis=-1,
                 keepdims=True)

    maskf = top2.astype(jnp.float32)
    ri = lax.broadcasted_iota(jnp.int32, (TB, TB), 0)
    ci = lax.broadcasted_iota(jnp.int32, (TB, TB), 1)
    tril = jnp.where(ri > ci, 1.0, 0.0)
    excl = jnp.dot(tril, maskf, preferred_element_type=jnp.float32)
    rank_mat = carry[...] + excl
    r1 = jnp.sum(jnp.where(eids == e1, rank_mat, 0.0), axis=1, keepdims=True)
    r2 = jnp.sum(jnp.where(eids == e2, rank_mat, 0.0), axis=1, keepdims=True)

    e12_ref[...] = jnp.concatenate([e1, e2], axis=1)
    r12_ref[...] = jnp.concatenate([r1, r2], axis=1).astype(jnp.int32)

    new_carry = carry[...] + jnp.sum(maskf, axis=0, keepdims=True)
    carry[...] = new_carry

    @pl.when(i == nb - 1)
    def _():
        cnt_ref[...] = new_carry.astype(jnp.int32)


def _gating(xf, gate_W, gb2):
    M, D = xf.shape
    return pl.pallas_call(
        _gate_body,
        grid=(M // TB,),
        in_specs=[
            pl.BlockSpec((TB, D), lambda i: (i, 0)),
            pl.BlockSpec((NE, D), lambda i: (0, 0)),
            pl.BlockSpec((1, NE), lambda i: (0, 0)),
        ],
        out_specs=[
            pl.BlockSpec((TB, 2), lambda i: (i, 0)),
            pl.BlockSpec((TB, 2), lambda i: (i, 0)),
            pl.BlockSpec((1, NE), lambda i: (0, 0)),
            pl.BlockSpec((TB, D // 2), lambda i: (i, 0)),
        ],
        out_shape=[
            jax.ShapeDtypeStruct((M, 2), jnp.int32),
            jax.ShapeDtypeStruct((M, 2), jnp.int32),
            jax.ShapeDtypeStruct((1, NE), jnp.int32),
            jax.ShapeDtypeStruct((M, D // 2), jnp.int32),
        ],
        scratch_shapes=[pltpu.VMEM((1, NE), jnp.float32)],
        compiler_params=pltpu.CompilerParams(
            dimension_semantics=("arbitrary",)),
    )(xf, gate_W, gb2)


def _make_dispatch(M, D, S_pad, n_chunk):
    mesh = plsc.VectorSubcoreMesh(core_axis_name="c", subcore_axis_name="s",
                                  num_cores=NC, num_subcores=NS)
    per_w = M // NW
    C = per_w // n_chunk
    D2 = D // 2

    @functools.partial(
        pl.kernel, mesh=mesh,
        out_type=jax.ShapeDtypeStruct((S_pad, D2), jnp.int32),
        scratch_types=[
            pltpu.VMEM((C, D2), jnp.int32),
            pltpu.VMEM((C, D2), jnp.int32),
            pltpu.VMEM((n_chunk, C), jnp.int32),
            pltpu.VMEM((n_chunk, C), jnp.int32),
            pltpu.SemaphoreType.DMA,
            pltpu.SemaphoreType.DMA,
            pltpu.SemaphoreType.DMA,
            pltpu.SemaphoreType.DMA,
        ],
    )
    def dispatch(x_hbm, s1_hbm, s2_hbm, xs_hbm, xb0, xb1, i1, i2,
                 sl0, sl1, ss0, ss1):
        wid = lax.axis_index("s") * NC + lax.axis_index("c")
        base = wid * per_w
        pltpu.sync_copy(s1_hbm.at[wid], i1)
        pltpu.sync_copy(s2_hbm.at[wid], i2)
        xb = [xb0, xb1]
        sem_l = [sl0, sl1]
        sem_s = [ss0, ss1]
        ld = [None] * n_chunk
        sc = [None] * (2 * n_chunk)
        ld[0] = pltpu.async_copy(x_hbm.at[pl.ds(base, C)], xb[0], sem_l[0])
        for c in range(n_chunk):
            b = c & 1
            if c + 1 < n_chunk:
                b2 = (c + 1) & 1
                if c >= 1:
                    sc[2 * (c - 1)].wait()
                    sc[2 * (c - 1) + 1].wait()
                ld[c + 1] = pltpu.async_copy(
                    x_hbm.at[pl.ds(base + (c + 1) * C, C)], xb[b2],
                    sem_l[b2])
            ld[c].wait()
            sc[2 * c] = pltpu.async_copy(xb[b], xs_hbm.at[i1.at[c]],
                                         sem_s[b])
            sc[2 * c + 1] = pltpu.async_copy(xb[b], xs_hbm.at[i2.at[c]],
                                             sem_s[b])
        for c in (n_chunk - 2, n_chunk - 1):
            sc[2 * c].wait()
            sc[2 * c + 1].wait()

    return dispatch


def _gmm_body(be_ref, xs_ref, w_ref, b_ref, y_ref):
    w = xs_ref[...]
    lo = lax.bitcast_convert_type(w & jnp.int32(-65536), jnp.float32)
    hi = lax.bitcast_convert_type(lax.shift_left(w, 16), jnp.float32)
    xb = jnp.concatenate([lo, hi], axis=1).astype(jnp.bfloat16)
    y = lax.dot_general(
        xb, w_ref[0],
        (((1,), (1,)), ((), ())), preferred_element_type=jnp.float32)
    y_ref[...] = y + b_ref[0]


def _grouped_matmul(xs, wq, eb3, be):
    S_pad, D2 = xs.shape
    O = wq.shape[1]
    nb = S_pad // BS
    grid_spec = pltpu.PrefetchScalarGridSpec(
        num_scalar_prefetch=1,
        grid=(nb,),
        in_specs=[
            pl.BlockSpec((BS, D2), lambda i, be: (i, 0)),
            pl.BlockSpec((1, O, 2 * D2), lambda i, be: (be[i], 0, 0)),
            pl.BlockSpec((1, 1, O), lambda i, be: (be[i], 0, 0)),
        ],
        out_specs=pl.BlockSpec((BS, O), lambda i, be: (i, 0)),
    )
    return pl.pallas_call(
        _gmm_body,
        grid_spec=grid_spec,
        out_shape=jax.ShapeDtypeStruct((S_pad, O), jnp.float32),
        compiler_params=pltpu.CompilerParams(
            dimension_semantics=("arbitrary",)),
    )(be, xs, wq, eb3)


def _make_combine(M, O, S_pad, n_chunk):
    mesh = plsc.VectorSubcoreMesh(core_axis_name="c", subcore_axis_name="s",
                                  num_cores=NC, num_subcores=NS)
    per_w = M // NW
    C = per_w // n_chunk
    L = 16

    @functools.partial(
        pl.kernel, mesh=mesh,
        out_type=jax.ShapeDtypeStruct((M, O), jnp.float32),
        scratch_types=[
            pltpu.VMEM((C, O), jnp.float32),
            pltpu.VMEM((C, O), jnp.float32),
            pltpu.VMEM((C, O), jnp.float32),
            pltpu.VMEM((C, O), jnp.float32),
            pltpu.VMEM((n_chunk, C), jnp.int32),
            pltpu.VMEM((n_chunk, C), jnp.int32),
            pltpu.SemaphoreType.DMA,
            pltpu.SemaphoreType.DMA,
            pltpu.SemaphoreType.DMA,
            pltpu.SemaphoreType.DMA,
        ],
    )
    def combine(ys_hbm, s1_hbm, s2_hbm, out_hbm, b1a, b2a, b1b, b2b, i1, i2,
                sg0, sg1, sw0, sw1):
        wid = lax.axis_index("s") * NC + lax.axis_index("c")
        base = wid * per_w
        pltpu.sync_copy(s1_hbm.at[wid], i1)
        pltpu.sync_copy(s2_hbm.at[wid], i2)
        b1 = [b1a, b1b]
        b2 = [b2a, b2b]
        sem_g = [sg0, sg1]
        sem_w = [sw0, sw1]
        g = [None] * (2 * n_chunk)
        w = [None] * n_chunk
        g[0] = pltpu.async_copy(ys_hbm.at[i1.at[0]], b1[0], sem_g[0])
        g[1] = pltpu.async_copy(ys_hbm.at[i2.at[0]], b2[0], sem_g[0])
        for c in range(n_chunk):
            s = c & 1
            if c + 1 < n_chunk:
                s2 = (c + 1) & 1
                if c >= 1:
                    w[c - 1].wait()
                g[2 * (c + 1)] = pltpu.async_copy(
                    ys_hbm.at[i1.at[c + 1]], b1[s2], sem_g[s2])
                g[2 * (c + 1) + 1] = pltpu.async_copy(
                    ys_hbm.at[i2.at[c + 1]], b2[s2], sem_g[s2])
            g[2 * c].wait()
            g[2 * c + 1].wait()

            def add_cols(j, _, s=s):
                for r in range(C):
                    b1[s][r, pl.ds(j * L, L)] = (
                        b1[s][r, pl.ds(j * L, L)]
                        + b2[s][r, pl.ds(j * L, L)])
                return 0

            lax.fori_loop(0, O // L, add_cols, 0)
            w[c] = pltpu.async_copy(b1[s], out_hbm.at[pl.ds(base + c * C, C)],
                                    sem_w[s])
        w[n_chunk - 1].wait()

    return combine


def kernel(x, gate_W, gate_b, expert_W, expert_b):
    orig_shape = x.shape
    D = x.shape[-1]
    M = x.size // D
    O = expert_W.shape[1]
    SL = D // 128
    xf = x.reshape(M, D)
    gb2 = gate_b.reshape(1, NE)
    wq = expert_W.astype(jnp.bfloat16)
    eb3 = expert_b.reshape(NE, 1, O)

    nb_max = M * 2 // BS + (NE - 1)
    S_pad = nb_max * BS
    n_disp, n_comb = 8, 32
    per_w = M // NW

    e12, r12, cnt, xq = _gating(xf, gate_W, gb2)

    counts = cnt[0]
    padded = ((counts + BS - 1) // BS) * BS
    poff = jnp.concatenate([jnp.zeros((1,), jnp.int32),
                            jnp.cumsum(padded)[:-1].astype(jnp.int32)])
    slot = jnp.take(poff, e12, axis=0) + r12
    s1 = slot[:, 0].reshape(NW, n_disp, per_w // n_disp)
    s2 = slot[:, 1].reshape(NW, n_disp, per_w // n_disp)
    starts = jnp.arange(nb_max, dtype=jnp.int32) * BS
    be = (jnp.sum(starts[:, None] >= poff[None, :], axis=1) - 1).astype(
        jnp.int32)
    be = jnp.clip(be, 0, NE - 1)

    xs_i = _make_dispatch(M, D, S_pad, n_disp)(xq, s1, s2)
    ys = _grouped_matmul(xs_i, wq, eb3, be)
    s1c = slot[:, 0].reshape(NW, n_comb, per_w // n_comb)
    s2c = slot[:, 1].reshape(NW, n_comb, per_w // n_comb)
    out = _make_combine(M, O, S_pad, n_comb)(ys, s1c, s2c)
    return out.reshape(orig_shape[:-1] + (O,))


# speedup vs baseline: 3.5276x; 1.0984x over previous
import functools

import jax
import jax.numpy as jnp
from jax import lax
from jax.experimental import pallas as pl
from jax.experimental.pallas import tpu as pltpu
from jax.experimental.pallas import tpu_sc as plsc

NE = 8
NC = 2
NS = 16
NW = NC * NS
TB = 1024
BS = 256


def _gate_body(x_ref, gw_ref, gb_ref, e12_ref, r12_ref, cnt_ref, xq_ref,
               carry):
    i = pl.program_id(0)
    nb = pl.num_programs(0)

    @pl.when(i == 0)
    def _():
        carry[...] = jnp.zeros_like(carry)

    x = x_ref[...]
    d2 = x.shape[1] // 2
    lo = x[:, :d2].astype(jnp.bfloat16).astype(jnp.float32)
    hi = x[:, d2:].astype(jnp.bfloat16).astype(jnp.float32)
    lo_i = lax.bitcast_convert_type(lo, jnp.int32)
    hi_i = lax.shift_right_logical(lax.bitcast_convert_type(hi, jnp.int32),
                                   16)
    xq_ref[...] = lo_i | hi_i
    logits = jnp.dot(x, gw_ref[...].T, preferred_element_type=jnp.float32)
    logits = logits + gb_ref[...]
    m1 = jnp.max(logits, axis=-1, keepdims=True)
    l2 = jnp.where(logits >= m1, -jnp.inf, logits)
    m2 = jnp.max(l2, axis=-1, keepdims=True)
    top2 = logits >= m2
    eids = lax.broadcasted_iota(jnp.int32, logits.shape, 1)
    big = jnp.int32(1 << 20)
    e1 = jnp.min(jnp.where(logits >= m1, eids, big), axis=-1, keepdims=True)
    e2 = jnp.min(jnp.where(top2 & (logits < m1), eids, big), axis=-1,
                 keepdims=True)

    maskf = top2.astype(jnp.float32)
    ri = lax.broadcasted_iota(jnp.int32, (TB, TB), 0)
    ci = lax.broadcasted_iota(jnp.int32, (TB, TB), 1)
    tril = jnp.where(ri > ci, 1.0, 0.0)
    excl = jnp.dot(tril, maskf, preferred_element_type=jnp.float32)
    rank_mat = carry[...] + excl
    r1 = jnp.sum(jnp.where(eids == e1, rank_mat, 0.0), axis=1, keepdims=True)
    r2 = jnp.sum(jnp.where(eids == e2, rank_mat, 0.0), axis=1, keepdims=True)

    e12_ref[...] = jnp.concatenate([e1, e2], axis=1)
    r12_ref[...] = jnp.concatenate([r1, r2], axis=1).astype(jnp.int32)

    new_carry = carry[...] + jnp.sum(maskf, axis=0, keepdims=True)
    carry[...] = new_carry

    @pl.when(i == nb - 1)
    def _():
        cnt_ref[...] = new_carry.astype(jnp.int32)


def _gating(xf, gate_W, gb2):
    M, D = xf.shape
    return pl.pallas_call(
        _gate_body,
        grid=(M // TB,),
        in_specs=[
            pl.BlockSpec((TB, D), lambda i: (i, 0)),
            pl.BlockSpec((NE, D), lambda i: (0, 0)),
            pl.BlockSpec((1, NE), lambda i: (0, 0)),
        ],
        out_specs=[
            pl.BlockSpec((TB, 2), lambda i: (i, 0)),
            pl.BlockSpec((TB, 2), lambda i: (i, 0)),
            pl.BlockSpec((1, NE), lambda i: (0, 0)),
            pl.BlockSpec((TB, D // 2), lambda i: (i, 0)),
        ],
        out_shape=[
            jax.ShapeDtypeStruct((M, 2), jnp.int32),
            jax.ShapeDtypeStruct((M, 2), jnp.int32),
            jax.ShapeDtypeStruct((1, NE), jnp.int32),
            jax.ShapeDtypeStruct((M, D // 2), jnp.int32),
        ],
        scratch_shapes=[pltpu.VMEM((1, NE), jnp.float32)],
        compiler_params=pltpu.CompilerParams(
            dimension_semantics=("arbitrary",)),
    )(xf, gate_W, gb2)


def _make_dispatch(M, D, S_pad, n_chunk):
    mesh = plsc.VectorSubcoreMesh(core_axis_name="c", subcore_axis_name="s",
                                  num_cores=NC, num_subcores=NS)
    per_w = M // NW
    C = per_w // n_chunk
    D2 = D // 2

    @functools.partial(
        pl.kernel, mesh=mesh,
        out_type=jax.ShapeDtypeStruct((S_pad, D2), jnp.int32),
        scratch_types=[
            pltpu.VMEM((C, D2), jnp.int32),
            pltpu.VMEM((C, D2), jnp.int32),
            pltpu.VMEM((n_chunk, C), jnp.int32),
            pltpu.VMEM((n_chunk, C), jnp.int32),
            pltpu.SemaphoreType.DMA,
            pltpu.SemaphoreType.DMA,
            pltpu.SemaphoreType.DMA,
            pltpu.SemaphoreType.DMA,
        ],
    )
    def dispatch(x_hbm, s1_hbm, s2_hbm, xs_hbm, xb0, xb1, i1, i2,
                 sl0, sl1, ss0, ss1):
        wid = lax.axis_index("s") * NC + lax.axis_index("c")
        base = wid * per_w
        pltpu.sync_copy(s1_hbm.at[wid], i1)
        pltpu.sync_copy(s2_hbm.at[wid], i2)
        xb = [xb0, xb1]
        sem_l = [sl0, sl1]
        sem_s = [ss0, ss1]
        ld = [None] * n_chunk
        sc = [None] * (2 * n_chunk)
        ld[0] = pltpu.async_copy(x_hbm.at[pl.ds(base, C)], xb[0], sem_l[0])
        for c in range(n_chunk):
            b = c & 1
            if c + 1 < n_chunk:
                b2 = (c + 1) & 1
                if c >= 1:
                    sc[2 * (c - 1)].wait()
                    sc[2 * (c - 1) + 1].wait()
                ld[c + 1] = pltpu.async_copy(
                    x_hbm.at[pl.ds(base + (c + 1) * C, C)], xb[b2],
                    sem_l[b2])
            ld[c].wait()
            sc[2 * c] = pltpu.async_copy(xb[b], xs_hbm.at[i1.at[c]],
                                         sem_s[b])
            sc[2 * c + 1] = pltpu.async_copy(xb[b], xs_hbm.at[i2.at[c]],
                                             sem_s[b])
        for c in (n_chunk - 2, n_chunk - 1):
            sc[2 * c].wait()
            sc[2 * c + 1].wait()

    return dispatch


def _gmm_body(be_ref, xs_ref, w_ref, b_ref, y_ref):
    w = xs_ref[...]
    lo = lax.bitcast_convert_type(w & jnp.int32(-65536), jnp.float32)
    hi = lax.bitcast_convert_type(lax.shift_left(w, 16), jnp.float32)
    xb = jnp.concatenate([lo, hi], axis=1)
    y = lax.dot_general(
        xb, w_ref[0],
        (((1,), (1,)), ((), ())), preferred_element_type=jnp.float32)
    y_ref[...] = y + b_ref[0]


def _grouped_matmul(xs, wq, eb3, be):
    S_pad, D2 = xs.shape
    O = wq.shape[1]
    nb = S_pad // BS
    grid_spec = pltpu.PrefetchScalarGridSpec(
        num_scalar_prefetch=1,
        grid=(nb,),
        in_specs=[
            pl.BlockSpec((BS, D2), lambda i, be: (i, 0)),
            pl.BlockSpec((1, O, 2 * D2), lambda i, be: (be[i], 0, 0)),
            pl.BlockSpec((1, 1, O), lambda i, be: (be[i], 0, 0)),
        ],
        out_specs=pl.BlockSpec((BS, O), lambda i, be: (i, 0)),
    )
    return pl.pallas_call(
        _gmm_body,
        grid_spec=grid_spec,
        out_shape=jax.ShapeDtypeStruct((S_pad, O), jnp.float32),
        compiler_params=pltpu.CompilerParams(
            dimension_semantics=("arbitrary",)),
    )(be, xs, wq, eb3)


def _make_combine(M, O, S_pad, n_chunk):
    mesh = plsc.VectorSubcoreMesh(core_axis_name="c", subcore_axis_name="s",
                                  num_cores=NC, num_subcores=NS)
    per_w = M // NW
    C = per_w // n_chunk
    L = 16

    @functools.partial(
        pl.kernel, mesh=mesh,
        out_type=jax.ShapeDtypeStruct((M, O), jnp.float32),
        scratch_types=[
            pltpu.VMEM((C, O), jnp.float32),
            pltpu.VMEM((C, O), jnp.float32),
            pltpu.VMEM((C, O), jnp.float32),
            pltpu.VMEM((C, O), jnp.float32),
            pltpu.VMEM((n_chunk, C), jnp.int32),
            pltpu.VMEM((n_chunk, C), jnp.int32),
            pltpu.SemaphoreType.DMA,
            pltpu.SemaphoreType.DMA,
            pltpu.SemaphoreType.DMA,
            pltpu.SemaphoreType.DMA,
        ],
    )
    def combine(ys_hbm, s1_hbm, s2_hbm, out_hbm, b1a, b2a, b1b, b2b, i1, i2,
                sg0, sg1, sw0, sw1):
        wid = lax.axis_index("s") * NC + lax.axis_index("c")
        base = wid * per_w
        pltpu.sync_copy(s1_hbm.at[wid], i1)
        pltpu.sync_copy(s2_hbm.at[wid], i2)
        b1 = [b1a, b1b]
        b2 = [b2a, b2b]
        sem_g = [sg0, sg1]
        sem_w = [sw0, sw1]
        g = [None] * (2 * n_chunk)
        w = [None] * n_chunk
        g[0] = pltpu.async_copy(ys_hbm.at[i1.at[0]], b1[0], sem_g[0])
        g[1] = pltpu.async_copy(ys_hbm.at[i2.at[0]], b2[0], sem_g[0])
        for c in range(n_chunk):
            s = c & 1
            if c + 1 < n_chunk:
                s2 = (c + 1) & 1
                if c >= 1:
                    w[c - 1].wait()
                g[2 * (c + 1)] = pltpu.async_copy(
                    ys_hbm.at[i1.at[c + 1]], b1[s2], sem_g[s2])
                g[2 * (c + 1) + 1] = pltpu.async_copy(
                    ys_hbm.at[i2.at[c + 1]], b2[s2], sem_g[s2])
            g[2 * c].wait()
            g[2 * c + 1].wait()

            def add_cols(j, _, s=s):
                for r in range(C):
                    b1[s][r, pl.ds(j * L, L)] = (
                        b1[s][r, pl.ds(j * L, L)]
                        + b2[s][r, pl.ds(j * L, L)])
                return 0

            lax.fori_loop(0, O // L, add_cols, 0)
            w[c] = pltpu.async_copy(b1[s], out_hbm.at[pl.ds(base + c * C, C)],
                                    sem_w[s])
        w[n_chunk - 1].wait()

    return combine


def kernel(x, gate_W, gate_b, expert_W, expert_b):
    orig_shape = x.shape
    D = x.shape[-1]
    M = x.size // D
    O = expert_W.shape[1]
    SL = D // 128
    xf = x.reshape(M, D)
    gb2 = gate_b.reshape(1, NE)
    eb3 = expert_b.reshape(NE, 1, O)

    nb_max = M * 2 // BS + (NE - 1)
    S_pad = nb_max * BS
    n_disp, n_comb = 8, 32
    per_w = M // NW

    e12, r12, cnt, xq = _gating(xf, gate_W, gb2)

    counts = cnt[0]
    padded = ((counts + BS - 1) // BS) * BS
    poff = jnp.concatenate([jnp.zeros((1,), jnp.int32),
                            jnp.cumsum(padded)[:-1].astype(jnp.int32)])
    slot = jnp.take(poff, e12, axis=0) + r12
    s1 = slot[:, 0].reshape(NW, n_disp, per_w // n_disp)
    s2 = slot[:, 1].reshape(NW, n_disp, per_w // n_disp)
    starts = jnp.arange(nb_max, dtype=jnp.int32) * BS
    be = (jnp.sum(starts[:, None] >= poff[None, :], axis=1) - 1).astype(
        jnp.int32)
    be = jnp.clip(be, 0, NE - 1)

    xs_i = _make_dispatch(M, D, S_pad, n_disp)(xq, s1, s2)
    ys = _grouped_matmul(xs_i, expert_W, eb3, be)
    s1c = slot[:, 0].reshape(NW, n_comb, per_w // n_comb)
    s2c = slot[:, 1].reshape(NW, n_comb, per_w // n_comb)
    out = _make_combine(M, O, S_pad, n_comb)(ys, s1c, s2c)
    return out.reshape(orig_shape[:-1] + (O,))
